# baseline (speedup 1.0000x reference)
import dataclasses

import jax
import jax.numpy as jnp
from jax.experimental import pallas as pl
from jax.experimental.pallas import tpu as pltpu
from jax.experimental.pallas import tpu_sc as plsc

_L = 4
_N = 256
_S = 16
_H = 8
_IB = 8
_JB = 128
_NC = 2
_NW = 32
_RPW = 2
_SC_ROWS_PER_L = _NW * _RPW // _L
_TC_ROWS = _N - _SC_ROWS_PER_L


def _tc_body(spt_ref, st_ref, spw_ref, edw_ref, out_ref, spw_scr, edw_scr):
    first = ((pl.program_id(0) == 0) & (pl.program_id(1) == 0)
             & (pl.program_id(2) == 0))

    @pl.when(first)
    def _prep():
        zeros = jnp.zeros((_H, 128), jnp.float32)
        spw_scr[...] = zeros
        edw_scr[...] = zeros
        spw_scr[:, pl.ds(0, 68)] = spw_ref[...]
        edw_scr[:, pl.ds(0, 32)] = edw_ref[...]

    etbs = [jnp.broadcast_to(edw_scr[h, :][None, :], (_IB, _JB))
            for h in range(_H)]
    stbs = [jnp.broadcast_to(spw_scr[h, :][None, :], (_IB, _JB))
            for h in range(_H)]
    cnt = jnp.zeros((_IB, _JB), jnp.int32)
    accs = [jnp.zeros((_IB, _JB), jnp.float32) for _ in range(_H)]
    for s in range(_S):
        x = spt_ref[0, :, s, :]
        m = x >= 0
        cnt = cnt + m.astype(jnp.int32)
        safe = jnp.where(m, x, 32)
        for h in range(_H):
            accs[h] = accs[h] + jnp.take_along_axis(
                etbs[h], safe, axis=1, mode="promise_in_bounds")
    recip = 1.0 / (cnt.astype(jnp.float32) + 1e-6)
    stv = st_ref[0]
    for h in range(_H):
        sp = jnp.take_along_axis(stbs[h], stv, axis=1,
                                 mode="promise_in_bounds")
        out_ref[0, h] = sp + accs[h] * recip


def _sc_body(spt_hbm, st_hbm, spw_hbm, edw_hbm, out_hbm,
             spt_v, st_v, tbl_s, tbl_e, out_v):
    wid = jax.lax.axis_index("s") * _NC + jax.lax.axis_index("c")
    lw = wid // 8
    k = wid % 8
    i0 = _TC_ROWS + k * _RPW
    ti = i0 // 8
    il0 = i0 % 8
    pltpu.sync_copy(spw_hbm, tbl_s)
    pltpu.sync_copy(edw_hbm, tbl_e)
    pltpu.sync_copy(spt_hbm.at[lw, pl.ds(i0, _RPW)], spt_v)
    pltpu.sync_copy(st_hbm.at[lw, ti, :, pl.ds(il0, _RPW)], st_v)

    hvecs = [jnp.full((16,), h, jnp.int32) for h in range(_H)]

    @pl.loop(0, _RPW)
    def _row(il):
        @pl.loop(0, 2)
        def _jhalf(jh):
            @pl.loop(0, 8)
            def _grp(jg):
                j0 = jg * 16
                negs = jnp.zeros((16,), jnp.int32)
                accs = [jnp.zeros((16,), jnp.float32) for _ in range(_H)]
                for s in range(_S):
                    xv = spt_v[il, s // 8, jh, s % 8, pl.ds(j0, 16)]
                    safe = jnp.bitwise_and(xv, 127)
                    negs = negs + jax.lax.shift_right_arithmetic(xv, 31)
                    for h in range(_H):
                        g = plsc.load_gather(tbl_e, [hvecs[h], safe])
                        accs[h] = accs[h] + g
                cnt = (_S + negs).astype(jnp.float32)
                recip = 1.0 / (cnt + 1e-6)
                stv = st_v[jh, il, pl.ds(j0, 16)]
                for h in range(_H):
                    sp = plsc.load_gather(tbl_s, [hvecs[h], stv])
                    out_v[h, jh, il, pl.ds(j0, 16)] = sp + accs[h] * recip

    pltpu.sync_copy(out_v, out_hbm.at[lw, :, ti - _TC_ROWS // 8, :,
                                      pl.ds(il0, _RPW)])


def kernel(spatial_encoder_weight, edge_dis_encoder_weight, spatial_types,
           shortest_path_types):
    spt_t = jnp.transpose(shortest_path_types, (0, 1, 3, 2))
    spw_t = spatial_encoder_weight.T
    edw_t = edge_dis_encoder_weight.T

    tc_out = pl.pallas_call(
        _tc_body,
        grid=(_L, _TC_ROWS // _IB, _N // _JB),
        in_specs=[
            pl.BlockSpec((1, _IB, _S, _JB), lambda l, i, j: (l, i, 0, j)),
            pl.BlockSpec((1, _IB, _JB), lambda l, i, j: (l, i, j)),
            pl.BlockSpec((_H, 68), lambda l, i, j: (0, 0)),
            pl.BlockSpec((_H, 32), lambda l, i, j: (0, 0)),
        ],
        out_specs=pl.BlockSpec((1, _H, _IB, _JB),
                               lambda l, i, j: (l, 0, i, j)),
        out_shape=jax.ShapeDtypeStruct((_L, _H, _TC_ROWS, _N), jnp.float32),
        scratch_shapes=[pltpu.VMEM((_H, 128), jnp.float32),
                        pltpu.VMEM((_H, 128), jnp.float32)],
    )(spt_t, spatial_types, spw_t, edw_t)

    spt6 = spt_t.reshape(_L, _N, 2, 8, 2, 128).transpose(0, 1, 2, 4, 3, 5)
    st6 = spatial_types.reshape(_L, 32, 8, 2, 128).transpose(0, 1, 3, 2, 4)
    spw_pad = jnp.pad(spw_t, ((0, 0), (0, 60)))
    edw_pad = jnp.pad(edw_t, ((0, 0), (0, 96)))

    cp = pltpu.CompilerParams()
    if "needs_layout_passes" in pltpu.CompilerParams.__dataclass_fields__:
        cp = dataclasses.replace(cp, needs_layout_passes=False)
    sc_mesh = plsc.VectorSubcoreMesh(core_axis_name="c", subcore_axis_name="s")
    sc_out_raw = pl.kernel(
        _sc_body,
        out_type=jax.ShapeDtypeStruct((_L, _H, 2, 2, 8, 128), jnp.float32),
        mesh=sc_mesh,
        compiler_params=cp,
        scratch_types=[
            pltpu.VMEM((_RPW, 2, 2, 8, 128), jnp.int32),
            pltpu.VMEM((2, _RPW, 128), jnp.int32),
            pltpu.VMEM((_H, 128), jnp.float32),
            pltpu.VMEM((_H, 128), jnp.float32),
            pltpu.VMEM((_H, 2, _RPW, 128), jnp.float32),
        ],
    )(spt6, st6, spw_pad, edw_pad)
    sc_tail = sc_out_raw.transpose(0, 1, 2, 4, 3, 5).reshape(
        _L, _H, _N - _TC_ROWS, _N)
    return jnp.concatenate([tc_out, sc_tail], axis=2)


# speedup vs baseline: 1.0993x; 1.0993x over previous
import jax
import jax.numpy as jnp
from jax.experimental import pallas as pl
from jax.experimental.pallas import tpu as pltpu

_L = 4
_N = 256
_S = 16
_H = 8
_IB = 8
_JB = 128


def _body(spt_ref, st_ref, spw_ref, edw_ref, out_ref, spw_scr, edw_scr):
    first = ((pl.program_id(0) == 0) & (pl.program_id(1) == 0)
             & (pl.program_id(2) == 0))

    @pl.when(first)
    def _prep():
        zeros = jnp.zeros((_H, 128), jnp.float32)
        spw_scr[...] = zeros
        edw_scr[...] = zeros
        spw_scr[:, pl.ds(0, 68)] = spw_ref[...]
        edw_scr[:, pl.ds(0, 32)] = edw_ref[...]

    etbs = [jnp.broadcast_to(edw_scr[h, :][None, :], (_IB, _JB))
            for h in range(_H)]
    stbs = [jnp.broadcast_to(spw_scr[h, :][None, :], (_IB, _JB))
            for h in range(_H)]
    cnt = jnp.zeros((_IB, _JB), jnp.int32)
    accs = [jnp.zeros((_IB, _JB), jnp.float32) for _ in range(_H)]
    for s in range(_S):
        x = spt_ref[0, :, s, :]
        m = x >= 0
        cnt = cnt + m.astype(jnp.int32)
        safe = jnp.where(m, x, 32)
        for h in range(_H):
            accs[h] = accs[h] + jnp.take_along_axis(
                etbs[h], safe, axis=1, mode="promise_in_bounds")
    recip = 1.0 / (cnt.astype(jnp.float32) + 1e-6)
    stv = st_ref[0]
    for h in range(_H):
        sp = jnp.take_along_axis(stbs[h], stv, axis=1,
                                 mode="promise_in_bounds")
        out_ref[0, h] = sp + accs[h] * recip


def kernel(spatial_encoder_weight, edge_dis_encoder_weight, spatial_types,
           shortest_path_types):
    spt_t = jnp.transpose(shortest_path_types, (0, 1, 3, 2))
    spw_t = spatial_encoder_weight.T
    edw_t = edge_dis_encoder_weight.T

    out = pl.pallas_call(
        _body,
        grid=(_L, _N // _IB, _N // _JB),
        in_specs=[
            pl.BlockSpec((1, _IB, _S, _JB), lambda l, i, j: (l, i, 0, j)),
            pl.BlockSpec((1, _IB, _JB), lambda l, i, j: (l, i, j)),
            pl.BlockSpec((_H, 68), lambda l, i, j: (0, 0)),
            pl.BlockSpec((_H, 32), lambda l, i, j: (0, 0)),
        ],
        out_specs=pl.BlockSpec((1, _H, _IB, _JB),
                               lambda l, i, j: (l, 0, i, j)),
        out_shape=jax.ShapeDtypeStruct((_L, _H, _N, _N), jnp.float32),
        scratch_shapes=[pltpu.VMEM((_H, 128), jnp.float32),
                        pltpu.VMEM((_H, 128), jnp.float32)],
    )(spt_t, spatial_types, spw_t, edw_t)
    return out


# speedup vs baseline: 1.6011x; 1.4565x over previous
import jax
import jax.numpy as jnp
from jax.experimental import pallas as pl
from jax.experimental.pallas import tpu as pltpu

_L = 4
_N = 256
_S = 16
_H = 8
_IB = 16
_JB = 128


def _body(spt_ref, st_ref, spw_ref, edw_ref, out_ref, spw_scr, edw_scr):
    first = ((pl.program_id(0) == 0) & (pl.program_id(1) == 0)
             & (pl.program_id(2) == 0))

    @pl.when(first)
    def _prep():
        zeros = jnp.zeros((_H, 128), jnp.float32)
        spw_scr[...] = zeros
        edw_scr[...] = zeros
        spw_scr[:, pl.ds(0, 68)] = spw_ref[...]
        edw_scr[:, pl.ds(0, 32)] = edw_ref[...]

    etbs = [jnp.broadcast_to(edw_scr[h, :][None, :], (_IB, _JB))
            for h in range(_H)]
    stbs = [jnp.broadcast_to(spw_scr[h, :][None, :], (_IB, _JB))
            for h in range(_H)]
    cnt = jnp.zeros((_IB, _JB), jnp.int32)
    accs = [jnp.zeros((_IB, _JB), jnp.float32) for _ in range(_H)]
    for s in range(_S):
        x = spt_ref[0, :, s, :]
        m = x >= 0
        cnt = cnt + m.astype(jnp.int32)
        safe = jnp.where(m, x, 32)
        for h in range(_H):
            accs[h] = accs[h] + jnp.take_along_axis(
                etbs[h], safe, axis=1, mode="promise_in_bounds")
    recip = 1.0 / (cnt.astype(jnp.float32) + 1e-6)
    stv = st_ref[0]
    for h in range(_H):
        sp = jnp.take_along_axis(stbs[h], stv, axis=1,
                                 mode="promise_in_bounds")
        out_ref[0, h] = sp + accs[h] * recip


def kernel(spatial_encoder_weight, edge_dis_encoder_weight, spatial_types,
           shortest_path_types):
    spt_t = jnp.transpose(shortest_path_types, (0, 1, 3, 2))
    spw_t = spatial_encoder_weight.T
    edw_t = edge_dis_encoder_weight.T

    out = pl.pallas_call(
        _body,
        grid=(_L, _N // _IB, _N // _JB),
        in_specs=[
            pl.BlockSpec((1, _IB, _S, _JB), lambda l, i, j: (l, i, 0, j)),
            pl.BlockSpec((1, _IB, _JB), lambda l, i, j: (l, i, j)),
            pl.BlockSpec((_H, 68), lambda l, i, j: (0, 0)),
            pl.BlockSpec((_H, 32), lambda l, i, j: (0, 0)),
        ],
        out_specs=pl.BlockSpec((1, _H, _IB, _JB),
                               lambda l, i, j: (l, 0, i, j)),
        out_shape=jax.ShapeDtypeStruct((_L, _H, _N, _N), jnp.float32),
        scratch_shapes=[pltpu.VMEM((_H, 128), jnp.float32),
                        pltpu.VMEM((_H, 128), jnp.float32)],
    )(spt_t, spatial_types, spw_t, edw_t)
    return out
